# baseline (speedup 1.0000x reference)
import functools

import jax
import jax.numpy as jnp
from jax.experimental import pallas as pl
from jax.experimental.pallas import tpu as pltpu

BN_EPS = 1e-5


def _seg_bn_relu(H, B, P):
    outs = []
    for j in range(P):
        h = H[j * B:(j + 1) * B]
        mean = jnp.mean(h, axis=0, keepdims=True)
        ms = jnp.mean(h * h, axis=0, keepdims=True)
        var = ms - mean * mean
        scale = jax.lax.rsqrt(var + BN_EPS)
        hb = h.astype(jnp.bfloat16)
        sb = scale.astype(jnp.bfloat16)
        tb = (mean * scale).astype(jnp.bfloat16)
        outs.append(jnp.maximum(hb * sb - tb, jnp.bfloat16(0.0)))
    return jnp.concatenate(outs, axis=0)


def _fused_kernel(x_ref, w_ref, bc_ref, out_ref, *, C, O, P):
    B = x_ref.shape[1]
    M = P * B
    X = x_ref[...].reshape(M, C).astype(jnp.bfloat16)
    H = jnp.dot(X, w_ref[0:C], preferred_element_type=jnp.float32)
    Hn = _seg_bn_relu(H, B, P)
    H = jnp.dot(Hn, w_ref[C:2 * C], preferred_element_type=jnp.float32)
    Hn = _seg_bn_relu(H, B, P)
    logits = jnp.dot(Hn, w_ref[2 * C:3 * C],
                     preferred_element_type=jnp.float32) + bc_ref[0:1, 0:O]
    E = jnp.exp(jnp.minimum(logits, 60.0))
    ones_rhs = jnp.ones((O, 128), jnp.bfloat16)
    S = jnp.dot(E.astype(jnp.bfloat16), ones_rhs,
                preferred_element_type=jnp.float32)[:, 0:1]
    out_ref[...] = (E * (1.0 / S)).reshape(P, B, O).astype(out_ref.dtype)


@jax.jit
def kernel(xs, packed):
    N, B, C = xs.shape
    O = C

    wslab = packed[0:3 * C, :].astype(jnp.bfloat16)
    bc_rows = packed[3 * C:3 * C + 8, :]

    P = 16 if N % 16 == 0 else 1
    grid_spec = pltpu.PrefetchScalarGridSpec(
        num_scalar_prefetch=0,
        grid=(N // P,),
        in_specs=[
            pl.BlockSpec((P, B, C), lambda i: (i, 0, 0)),
            pl.BlockSpec(wslab.shape, lambda i: (0, 0)),
            pl.BlockSpec(bc_rows.shape, lambda i: (0, 0)),
        ],
        out_specs=pl.BlockSpec((P, B, O), lambda i: (i, 0, 0)),
    )
    flops = 2 * N * B * C * (2 * C + O)
    cost = pl.CostEstimate(
        flops=flops,
        transcendentals=N * B * (O + 2 * C),
        bytes_accessed=4 * N * B * (C + O) + 2 * wslab.size,
    )
    return pl.pallas_call(
        functools.partial(_fused_kernel, C=C, O=O, P=P),
        out_shape=jax.ShapeDtypeStruct((N, B, O), jnp.float32),
        grid_spec=grid_spec,
        compiler_params=pltpu.CompilerParams(dimension_semantics=("parallel",)),
        cost_estimate=cost,
    )(xs, wslab, bc_rows)


# speedup vs baseline: 1.0586x; 1.0586x over previous
import functools

import jax
import jax.numpy as jnp
from jax.experimental import pallas as pl
from jax.experimental.pallas import tpu as pltpu

BN_EPS = 1e-5


def _seg_bn_relu(H, B, P):
    outs = []
    for j in range(P):
        h = H[j * B:(j + 1) * B]
        mean = jnp.mean(h, axis=0, keepdims=True)
        ms = jnp.mean(h * h, axis=0, keepdims=True)
        var = ms - mean * mean
        scale = jax.lax.rsqrt(var + BN_EPS)
        hb = h.astype(jnp.bfloat16)
        sb = scale.astype(jnp.bfloat16)
        tb = (mean * scale).astype(jnp.bfloat16)
        outs.append(jnp.maximum(hb * sb - tb, jnp.bfloat16(0.0)))
    return jnp.concatenate(outs, axis=0)


def _fused_kernel(x_ref, w_ref, bc_ref, out_ref, *, C, O, P):
    B = x_ref.shape[1]
    M = P * B
    X = x_ref[...].reshape(M, C).astype(jnp.bfloat16)
    H = jnp.dot(X, w_ref[0:C], preferred_element_type=jnp.float32)
    Hn = _seg_bn_relu(H, B, P)
    H = jnp.dot(Hn, w_ref[C:2 * C], preferred_element_type=jnp.float32)
    Hn = _seg_bn_relu(H, B, P)
    logits = jnp.dot(Hn, w_ref[2 * C:3 * C],
                     preferred_element_type=jnp.float32) + bc_ref[0:1, 0:O]
    E = jnp.exp(jnp.minimum(logits, 60.0))
    ones_rhs = jnp.ones((O, 128), jnp.bfloat16)
    S = jnp.dot(E.astype(jnp.bfloat16), ones_rhs,
                preferred_element_type=jnp.float32)[:, 0:1]
    out_ref[...] = (E * (1.0 / S)).reshape(P, B, O).astype(out_ref.dtype)


@jax.jit
def kernel(xs, packed):
    N, B, C = xs.shape
    O = C

    wslab = packed[0:3 * C, :].astype(jnp.bfloat16)
    bc_rows = packed[3 * C:3 * C + 8, :]

    P = 8 if N % 8 == 0 else 1
    grid_spec = pltpu.PrefetchScalarGridSpec(
        num_scalar_prefetch=0,
        grid=(N // P,),
        in_specs=[
            pl.BlockSpec((P, B, C), lambda i: (i, 0, 0)),
            pl.BlockSpec(wslab.shape, lambda i: (0, 0)),
            pl.BlockSpec(bc_rows.shape, lambda i: (0, 0)),
        ],
        out_specs=pl.BlockSpec((P, B, O), lambda i: (i, 0, 0)),
    )
    flops = 2 * N * B * C * (2 * C + O)
    cost = pl.CostEstimate(
        flops=flops,
        transcendentals=N * B * (O + 2 * C),
        bytes_accessed=4 * N * B * (C + O) + 2 * wslab.size,
    )
    return pl.pallas_call(
        functools.partial(_fused_kernel, C=C, O=O, P=P),
        out_shape=jax.ShapeDtypeStruct((N, B, O), jnp.float32),
        grid_spec=grid_spec,
        compiler_params=pltpu.CompilerParams(dimension_semantics=("parallel",)),
        cost_estimate=cost,
    )(xs, wslab, bc_rows)
